# baseline (speedup 1.0000x reference)
import jax
import jax.numpy as jnp
from jax.experimental import pallas as pl
from jax.experimental.pallas import tpu as pltpu


def _se_kernel(x_ref, w1s_ref, b1_ref, w2_ref, b2_ref, o_ref):
    xs = x_ref[...]
    m = jnp.sum(xs, axis=1, dtype=jnp.float32)
    h = jax.lax.dot_general(m, w1s_ref[...], (((1,), (1,)), ((), ())),
                            preferred_element_type=jnp.float32)
    h = jnp.maximum(h + b1_ref[...], 0.0)
    g = jax.lax.dot_general(h, w2_ref[...], (((1,), (1,)), ((), ())),
                            preferred_element_type=jnp.float32)
    g = jax.nn.sigmoid(g + b2_ref[...])
    o_ref[...] = (xs * g.astype(xs.dtype)[:, None, :]).astype(o_ref.dtype)


def kernel(x, w1, b1, w2, b2):
    n, c, h, w = x.shape
    hw = h * w
    cr = w1.shape[0]

    xv = jnp.transpose(x, (0, 2, 3, 1)).reshape(n, hw, c)

    w1s = (w1 * (1.0 / hw)).astype(jnp.float32)
    b1r = b1.reshape(1, cr).astype(jnp.float32)
    b2r = b2.reshape(1, c).astype(jnp.float32)

    tn = 2 if n % 2 == 0 else 1
    out = pl.pallas_call(
        _se_kernel,
        out_shape=jax.ShapeDtypeStruct((n, hw, c), x.dtype),
        grid=(n // tn,),
        in_specs=[
            pl.BlockSpec((tn, hw, c), lambda i: (i, 0, 0)),
            pl.BlockSpec((cr, c), lambda i: (0, 0)),
            pl.BlockSpec((1, cr), lambda i: (0, 0)),
            pl.BlockSpec((c, cr), lambda i: (0, 0)),
            pl.BlockSpec((1, c), lambda i: (0, 0)),
        ],
        out_specs=pl.BlockSpec((tn, hw, c), lambda i: (i, 0, 0)),
        compiler_params=pltpu.CompilerParams(
            dimension_semantics=("parallel",),
            vmem_limit_bytes=56 * 1024 * 1024,
        ),
    )(xv, w1s, b1r, w2, b2r)

    return out.reshape(n, h, w, c).transpose(0, 3, 1, 2)
